# baseline (speedup 1.0000x reference)
import jax
import jax.numpy as jnp
from jax import lax
from jax.experimental import pallas as pl
from jax.experimental.pallas import tpu as pltpu
from jax.experimental.pallas import tpu_sc as plsc

HIDDEN = 64
VOCAB = 1000000
NUM_CORES = 2
NUM_SUBCORES = 16
NW = NUM_CORES * NUM_SUBCORES
B = 4096
T = 200

MB_COLS = 256
N_MB = VOCAB // MB_COLS
MB_PER_W = 123
TAIL_C0 = N_MB * MB_COLS
H1 = 245 * 2048

_MESH = plsc.VectorSubcoreMesh(core_axis_name="c", subcore_axis_name="s")


def _wid():
    return lax.axis_index("s") * NUM_CORES + lax.axis_index("c")


_IOTA = None


def _transpose_to_pairs(x_v, p_v, npairs):
    ncols = 2 * npairs
    nq = ncols // 16
    iot = lax.iota(jnp.int32, 16)
    rows = [lax.shift_right_logical(16 * q + iot, 1) for q in range(nq)]
    cols = [lax.shift_left(lax.bitwise_and(16 * q + iot, 1), 6)
            for q in range(nq)]

    @plsc.parallel_loop(0, HIDDEN, unroll=4)
    def _hstep(h):
        hv = jnp.full((16,), 0, jnp.int32) + h
        for q in range(nq):
            v = x_v[h, pl.ds(16 * q, 16)]
            plsc.store_scatter(p_v, [rows[q], cols[q] + hv], v)


def _k1_body(tab_t, tabP, x0, x1, x2, p0, p1, p2, xt,
             rs0, rs1, rs2, ws0, ws1, ws2):
    wid = _wid()
    base = wid * MB_PER_W
    xs = (x0, x1, x2)
    ps = (p0, p1, p2)
    rss = (rs0, rs1, rs2)
    wss = (ws0, ws1, ws2)

    def mb(g):
        return jnp.minimum(base + g, N_MB - 1)

    def read(g, b):
        pltpu.async_copy(tab_t.at[:, pl.ds(mb(g) * MB_COLS, MB_COLS)],
                         xs[b], rss[b])

    def write(g, b):
        pltpu.async_copy(ps[b], tabP.at[pl.ds(mb(g) * (MB_COLS // 2),
                                              MB_COLS // 2), :], wss[b])

    for b in range(3):
        read(b, b)

    def step(g, b, wait_w):
        pltpu.make_async_copy(tab_t.at[:, pl.ds(0, MB_COLS)], xs[b],
                              rss[b]).wait()
        if wait_w:
            pltpu.make_async_copy(ps[b], tabP.at[pl.ds(0, MB_COLS // 2), :],
                                  wss[b]).wait()
        _transpose_to_pairs(xs[b], ps[b], 128)
        read(g + 3, b)
        write(g, b)

    step(0, 0, False)
    step(1, 1, False)
    step(2, 2, False)

    def tri(tt, carry):
        g = 3 * tt
        step(g, 0, True)
        step(g + 1, 1, True)
        step(g + 2, 2, True)
        return carry

    lax.fori_loop(1, MB_PER_W // 3, tri, 0)

    for b in range(3):
        pltpu.make_async_copy(tab_t.at[:, pl.ds(0, MB_COLS)], xs[b],
                              rss[b]).wait()
        pltpu.make_async_copy(ps[b], tabP.at[pl.ds(0, MB_COLS // 2), :],
                              wss[b]).wait()

    @pl.when(wid == NW - 1)
    def _tail():
        c0 = TAIL_C0 + lax.axis_index("c") * 0
        pltpu.sync_copy(tab_t.at[:, pl.ds(c0, 128)], xt)
        _transpose_to_pairs(xt, p0, 32)
        pltpu.sync_copy(p0.at[pl.ds(0, 32), :],
                        tabP.at[pl.ds(TAIL_C0 // 2, 32), :])


def _extract_half(g_v, c0, o_v):
    iot = lax.iota(jnp.int32, 16)
    rows = [lax.shift_right_logical(16 * q + iot, 3) for q in range(4)]
    cols = [lax.shift_left(lax.bitwise_and(16 * q + iot, 7), 7)
            for q in range(4)]

    @plsc.parallel_loop(0, 256, unroll=8)
    def _cstep(c):
        jv = jnp.full((16,), 0, jnp.int32) + lax.shift_right_logical(c, 7)
        cv = jnp.full((16,), 0, jnp.int32) + lax.bitwise_and(c, 127)
        for q in range(4):
            v = g_v[c0 + c, pl.ds(16 * q, 16)]
            plsc.store_scatter(o_v, [rows[q], jv, cols[q] + cv], v)


def _k2_body(ids_flat, tabv, out5, idx0, idx1, g0, g1, oA, oB,
             gs0, gs1, wsA, wsB):
    wid = _wid()
    g_lo = wid * 50
    gs = (g0, g1)
    idxs = (idx0, idx1)
    gss = (gs0, gs1)

    def stage_and_gather(G, b):
        Gc = jnp.minimum(G, g_lo + 49)
        pltpu.sync_copy(ids_flat.at[pl.ds(512 * Gc, 512)], idxs[b])
        idx_ref = idxs[b]

        @plsc.parallel_loop(0, 512, step=16, unroll=8)
        def _remap(i):
            v = idx_ref[pl.ds(i, 16)]
            adj = jnp.where(v >= H1, 2 * H1 - 1, 0)
            idx_ref[pl.ds(i, 16)] = lax.shift_left(v, 1) - adj

        pltpu.async_copy(tabv.at[idxs[b]], gs[b], gss[b])

    def wslice(G, half):
        t = lax.shift_right_logical(G, 3)
        j0 = 4 * lax.bitwise_and(G, 7) + 2 * half
        return out5.at[t, :, pl.ds(j0, 2), :]

    stage_and_gather(g_lo, 0)
    stage_and_gather(g_lo + 1, 1)

    def step(G, b, wait_w):
        pltpu.make_async_copy(tabv.at[idxs[b]], gs[b], gss[b]).wait()
        if wait_w:
            pltpu.make_async_copy(oA, wslice(g_lo, 0), wsA).wait()
        _extract_half(gs[b], 0, oA)
        pltpu.async_copy(oA, wslice(G, 0), wsA)
        if wait_w:
            pltpu.make_async_copy(oB, wslice(g_lo, 1), wsB).wait()
        _extract_half(gs[b], 256, oB)
        pltpu.async_copy(oB, wslice(G, 1), wsB)
        stage_and_gather(G + 2, b)

    step(g_lo, 0, False)
    step(g_lo + 1, 1, True)

    def pairg(kk, carry):
        G0 = g_lo + 2 * kk
        step(G0, 0, True)
        step(G0 + 1, 1, True)
        return carry

    lax.fori_loop(1, 25, pairg, 0)

    for b in range(2):
        pltpu.make_async_copy(tabv.at[idxs[b]], gs[b], gss[b]).wait()
    pltpu.make_async_copy(oA, wslice(g_lo, 0), wsA).wait()
    pltpu.make_async_copy(oB, wslice(g_lo, 1), wsB).wait()


def _k1_tc_body(x1_ref, x2_ref, y_ref):
    y_ref[:, 0:64] = x1_ref[...].T
    y_ref[:, 64:128] = x2_ref[...].T


def _k1b_body(x1_ref, x2_ref, tin_ref, y_ref):
    del tin_ref
    y_ref[:, 0:64] = x1_ref[...].T
    y_ref[:, 64:128] = x2_ref[...].T


@jax.jit
def _embed(ids, table):
    tab_t = table.T

    k1 = pl.pallas_call(
        _k1_tc_body,
        grid=(H1 // 2048,),
        in_specs=[
            pl.BlockSpec((HIDDEN, 2048), lambda i: (0, i)),
            pl.BlockSpec((HIDDEN, 2048),
                         lambda i: (0, jnp.minimum(i + H1 // 2048,
                                                   VOCAB // 2048 - 1))),
        ],
        out_specs=pl.BlockSpec((2048, 128), lambda i: (i, 0)),
        out_shape=jax.ShapeDtypeStruct((H1, 128), jnp.float32),
    )
    tabP = k1(tab_t, tab_t)

    k1b = pl.pallas_call(
        _k1b_body,
        grid=(1,),
        in_specs=[
            pl.BlockSpec((HIDDEN, 1024), lambda i: (0, 497664 // 1024)),
            pl.BlockSpec((HIDDEN, 1024), lambda i: (0, 999424 // 1024)),
            pl.BlockSpec((8, 128), lambda i: (0, 0)),
        ],
        out_specs=pl.BlockSpec((1024, 128), lambda i: (497664 // 1024, 0)),
        out_shape=jax.ShapeDtypeStruct((H1, 128), jnp.float32),
        input_output_aliases={2: 0},
    )
    tabP = k1b(tab_t, tab_t, tabP)
    tabv = tabP.reshape(2 * H1, HIDDEN)

    ids_flat = ids.T.reshape(-1)

    k2 = pl.kernel(
        _k2_body,
        mesh=_MESH,
        out_type=jax.ShapeDtypeStruct((T, 8, 32, 1024), jnp.float32),
        scratch_types=[
            pltpu.VMEM((512,), jnp.int32),
            pltpu.VMEM((512,), jnp.int32),
            pltpu.VMEM((512, HIDDEN), jnp.float32),
            pltpu.VMEM((512, HIDDEN), jnp.float32),
            pltpu.VMEM((8, 2, 1024), jnp.float32),
            pltpu.VMEM((8, 2, 1024), jnp.float32),
        ] + [pltpu.SemaphoreType.DMA] * 4,
        compiler_params=pltpu.CompilerParams(use_tc_tiling_on_sc=False,
                                             needs_layout_passes=False,
                                             disable_bounds_checks=True),
    )
    out5 = k2(ids_flat, tabv)
    out6 = out5.reshape(T, 8, 32, 8, 128)
    return jnp.transpose(out6, (2, 4, 0, 1, 3)).reshape(B, T, HIDDEN)


def kernel(input_ids, embed_tokens_weight):
    return _embed(input_ids.astype(jnp.int32), embed_tokens_weight)


# speedup vs baseline: 1.0330x; 1.0330x over previous
import jax
import jax.numpy as jnp
from jax import lax
from jax.experimental import pallas as pl
from jax.experimental.pallas import tpu as pltpu
from jax.experimental.pallas import tpu_sc as plsc

HIDDEN = 64
VOCAB = 1000000
NUM_CORES = 2
NUM_SUBCORES = 16
NW = NUM_CORES * NUM_SUBCORES
B = 4096
T = 200

MB_COLS = 256
N_MB = VOCAB // MB_COLS
MB_PER_W = 123
TAIL_C0 = N_MB * MB_COLS
H1 = 245 * 2048

_MESH = plsc.VectorSubcoreMesh(core_axis_name="c", subcore_axis_name="s")


def _wid():
    return lax.axis_index("s") * NUM_CORES + lax.axis_index("c")


_IOTA = None


def _transpose_to_pairs(x_v, p_v, npairs):
    ncols = 2 * npairs
    nq = ncols // 16
    iot = lax.iota(jnp.int32, 16)
    rows = [lax.shift_right_logical(16 * q + iot, 1) for q in range(nq)]
    cols = [lax.shift_left(lax.bitwise_and(16 * q + iot, 1), 6)
            for q in range(nq)]

    @plsc.parallel_loop(0, HIDDEN, unroll=4)
    def _hstep(h):
        hv = jnp.full((16,), 0, jnp.int32) + h
        for q in range(nq):
            v = x_v[h, pl.ds(16 * q, 16)]
            plsc.store_scatter(p_v, [rows[q], cols[q] + hv], v)


def _k1_body(tab_t, tabP, x0, x1, x2, p0, p1, p2, xt,
             rs0, rs1, rs2, ws0, ws1, ws2):
    wid = _wid()
    base = wid * MB_PER_W
    xs = (x0, x1, x2)
    ps = (p0, p1, p2)
    rss = (rs0, rs1, rs2)
    wss = (ws0, ws1, ws2)

    def mb(g):
        return jnp.minimum(base + g, N_MB - 1)

    def read(g, b):
        pltpu.async_copy(tab_t.at[:, pl.ds(mb(g) * MB_COLS, MB_COLS)],
                         xs[b], rss[b])

    def write(g, b):
        pltpu.async_copy(ps[b], tabP.at[pl.ds(mb(g) * (MB_COLS // 2),
                                              MB_COLS // 2), :], wss[b])

    for b in range(3):
        read(b, b)

    def step(g, b, wait_w):
        pltpu.make_async_copy(tab_t.at[:, pl.ds(0, MB_COLS)], xs[b],
                              rss[b]).wait()
        if wait_w:
            pltpu.make_async_copy(ps[b], tabP.at[pl.ds(0, MB_COLS // 2), :],
                                  wss[b]).wait()
        _transpose_to_pairs(xs[b], ps[b], 128)
        read(g + 3, b)
        write(g, b)

    step(0, 0, False)
    step(1, 1, False)
    step(2, 2, False)

    def tri(tt, carry):
        g = 3 * tt
        step(g, 0, True)
        step(g + 1, 1, True)
        step(g + 2, 2, True)
        return carry

    lax.fori_loop(1, MB_PER_W // 3, tri, 0)

    for b in range(3):
        pltpu.make_async_copy(tab_t.at[:, pl.ds(0, MB_COLS)], xs[b],
                              rss[b]).wait()
        pltpu.make_async_copy(ps[b], tabP.at[pl.ds(0, MB_COLS // 2), :],
                              wss[b]).wait()

    @pl.when(wid == NW - 1)
    def _tail():
        c0 = TAIL_C0 + lax.axis_index("c") * 0
        pltpu.sync_copy(tab_t.at[:, pl.ds(c0, 128)], xt)
        _transpose_to_pairs(xt, p0, 32)
        pltpu.sync_copy(p0.at[pl.ds(0, 32), :],
                        tabP.at[pl.ds(TAIL_C0 // 2, 32), :])


def _extract2(g_v, o_v):
    iot = lax.iota(jnp.int32, 16)
    rows = [lax.shift_right_logical(16 * q + iot, 3) for q in range(4)]
    cols = [lax.shift_left(lax.bitwise_and(16 * q + iot, 7), 7)
            for q in range(4)]
    for tt in range(2):
        ttv = jnp.full((16,), tt, jnp.int32)

        @plsc.parallel_loop(0, 128, unroll=8)
        def _cstep(c):
            cv = jnp.full((16,), 0, jnp.int32) + c
            for q in range(4):
                v = g_v[128 * tt + c, pl.ds(16 * q, 16)]
                plsc.store_scatter(o_v, [ttv, rows[q], cols[q] + cv], v)


def _k2_body(ids4, tabv, out5, ids_v, g0, g1, o0, o1,
             gs0, gs1, ws0, ws1):
    wid = _wid()
    pltpu.sync_copy(ids4.at[wid], ids_v)

    @plsc.parallel_loop(0, 25600, step=16, unroll=8)
    def _remap(i):
        v = ids_v[pl.ds(i, 16)]
        adj = jnp.where(v >= H1, 2 * H1 - 1, 0)
        ids_v[pl.ds(i, 16)] = lax.shift_left(v, 1) - adj

    gs = (g0, g1)
    os_ = (o0, o1)
    gss = (gs0, gs1)
    wss = (ws0, ws1)
    nk = T // 2

    def gather(k, b):
        kc = jnp.minimum(k, nk - 1)
        pltpu.async_copy(tabv.at[ids_v.at[pl.ds(256 * kc, 256)]],
                         gs[b], gss[b])

    def write(k, b):
        pltpu.async_copy(os_[b], out5.at[pl.ds(2 * k, 2), :, wid, :], wss[b])

    gather(0, 0)
    gather(1, 1)

    def step(k, b, wait_w):
        pltpu.make_async_copy(tabv.at[ids_v.at[pl.ds(0, 256)]], gs[b],
                              gss[b]).wait()
        if wait_w:
            pltpu.make_async_copy(os_[b], out5.at[pl.ds(0, 2), :, wid, :],
                                  wss[b]).wait()
        _extract2(gs[b], os_[b])
        gather(k + 2, b)
        write(k, b)

    step(0, 0, False)
    step(1, 1, False)

    def pairk(kk, carry):
        k0 = 2 * kk
        step(k0, 0, True)
        step(k0 + 1, 1, True)
        return carry

    lax.fori_loop(1, nk // 2, pairk, 0)

    for b in range(2):
        pltpu.make_async_copy(tabv.at[ids_v.at[pl.ds(0, 256)]], gs[b],
                              gss[b]).wait()
        pltpu.make_async_copy(os_[b], out5.at[pl.ds(0, 2), :, wid, :],
                              wss[b]).wait()


def _k1_tc_body(x1_ref, x2_ref, y_ref):
    y_ref[:, 0:64] = x1_ref[...].T
    y_ref[:, 64:128] = x2_ref[...].T


def _k1b_body(x1_ref, x2_ref, tin_ref, y_ref):
    del tin_ref
    y_ref[:, 0:64] = x1_ref[...].T
    y_ref[:, 64:128] = x2_ref[...].T


@jax.jit
def _embed(ids, table):
    tab_t = table.T

    k1 = pl.pallas_call(
        _k1_tc_body,
        grid=(H1 // 2048,),
        in_specs=[
            pl.BlockSpec((HIDDEN, 2048), lambda i: (0, i)),
            pl.BlockSpec((HIDDEN, 2048),
                         lambda i: (0, jnp.minimum(i + H1 // 2048,
                                                   VOCAB // 2048 - 1))),
        ],
        out_specs=pl.BlockSpec((2048, 128), lambda i: (i, 0)),
        out_shape=jax.ShapeDtypeStruct((H1, 128), jnp.float32),
    )
    tabP = k1(tab_t, tab_t)

    k1b = pl.pallas_call(
        _k1b_body,
        grid=(1,),
        in_specs=[
            pl.BlockSpec((HIDDEN, 1024), lambda i: (0, 497664 // 1024)),
            pl.BlockSpec((HIDDEN, 1024), lambda i: (0, 999424 // 1024)),
            pl.BlockSpec((8, 128), lambda i: (0, 0)),
        ],
        out_specs=pl.BlockSpec((1024, 128), lambda i: (497664 // 1024, 0)),
        out_shape=jax.ShapeDtypeStruct((H1, 128), jnp.float32),
        input_output_aliases={2: 0},
    )
    tabP = k1b(tab_t, tab_t, tabP)
    tabv = tabP.reshape(2 * H1, HIDDEN)

    ids4 = ids.T.reshape(T, NW, 128).transpose(1, 0, 2).reshape(NW, 25600)

    k2 = pl.kernel(
        _k2_body,
        mesh=_MESH,
        out_type=jax.ShapeDtypeStruct((T, 8, 32, 1024), jnp.float32),
        scratch_types=[
            pltpu.VMEM((25600,), jnp.int32),
        ] + [pltpu.VMEM((256, HIDDEN), jnp.float32)] * 2
          + [pltpu.VMEM((2, 8, 1024), jnp.float32)] * 2
          + [pltpu.SemaphoreType.DMA] * 4,
        compiler_params=pltpu.CompilerParams(use_tc_tiling_on_sc=False,
                                             needs_layout_passes=False,
                                             disable_bounds_checks=True),
    )
    out5 = k2(ids4, tabv)
    out6 = out5.reshape(T, 8, 32, 8, 128)
    return jnp.transpose(out6, (2, 4, 0, 1, 3)).reshape(B, T, HIDDEN)


def kernel(input_ids, embed_tokens_weight):
    return _embed(input_ids.astype(jnp.int32), embed_tokens_weight)


# speedup vs baseline: 1.0364x; 1.0033x over previous
import jax
import jax.numpy as jnp
from jax import lax
from jax.experimental import pallas as pl
from jax.experimental.pallas import tpu as pltpu
from jax.experimental.pallas import tpu_sc as plsc

HIDDEN = 64
VOCAB = 1000000
NUM_CORES = 2
NUM_SUBCORES = 16
NW = NUM_CORES * NUM_SUBCORES
B = 4096
T = 200

MB_COLS = 256
N_MB = VOCAB // MB_COLS
MB_PER_W = 123
TAIL_C0 = N_MB * MB_COLS
H1 = 245 * 2048

def _mesh():
    return plsc.VectorSubcoreMesh(core_axis_name="c", subcore_axis_name="s")


def _wid():
    return lax.axis_index("s") * NUM_CORES + lax.axis_index("c")


_IOTA = None


def _transpose_to_pairs(x_v, p_v, npairs):
    ncols = 2 * npairs
    nq = ncols // 16
    iot = lax.iota(jnp.int32, 16)
    rows = [lax.shift_right_logical(16 * q + iot, 1) for q in range(nq)]
    cols = [lax.shift_left(lax.bitwise_and(16 * q + iot, 1), 6)
            for q in range(nq)]

    @plsc.parallel_loop(0, HIDDEN, unroll=4)
    def _hstep(h):
        hv = jnp.full((16,), 0, jnp.int32) + h
        for q in range(nq):
            v = x_v[h, pl.ds(16 * q, 16)]
            plsc.store_scatter(p_v, [rows[q], cols[q] + hv], v)


def _k1_body(tab_t, tabP, x0, x1, x2, p0, p1, p2, xt,
             rs0, rs1, rs2, ws0, ws1, ws2):
    wid = _wid()
    base = wid * MB_PER_W
    xs = (x0, x1, x2)
    ps = (p0, p1, p2)
    rss = (rs0, rs1, rs2)
    wss = (ws0, ws1, ws2)

    def mb(g):
        return jnp.minimum(base + g, N_MB - 1)

    def read(g, b):
        pltpu.async_copy(tab_t.at[:, pl.ds(mb(g) * MB_COLS, MB_COLS)],
                         xs[b], rss[b])

    def write(g, b):
        pltpu.async_copy(ps[b], tabP.at[pl.ds(mb(g) * (MB_COLS // 2),
                                              MB_COLS // 2), :], wss[b])

    for b in range(3):
        read(b, b)

    def step(g, b, wait_w):
        pltpu.make_async_copy(tab_t.at[:, pl.ds(0, MB_COLS)], xs[b],
                              rss[b]).wait()
        if wait_w:
            pltpu.make_async_copy(ps[b], tabP.at[pl.ds(0, MB_COLS // 2), :],
                                  wss[b]).wait()
        _transpose_to_pairs(xs[b], ps[b], 128)
        read(g + 3, b)
        write(g, b)

    step(0, 0, False)
    step(1, 1, False)
    step(2, 2, False)

    def tri(tt, carry):
        g = 3 * tt
        step(g, 0, True)
        step(g + 1, 1, True)
        step(g + 2, 2, True)
        return carry

    lax.fori_loop(1, MB_PER_W // 3, tri, 0)

    for b in range(3):
        pltpu.make_async_copy(tab_t.at[:, pl.ds(0, MB_COLS)], xs[b],
                              rss[b]).wait()
        pltpu.make_async_copy(ps[b], tabP.at[pl.ds(0, MB_COLS // 2), :],
                              wss[b]).wait()

    @pl.when(wid == NW - 1)
    def _tail():
        c0 = TAIL_C0 + lax.axis_index("c") * 0
        pltpu.sync_copy(tab_t.at[:, pl.ds(c0, 128)], xt)
        _transpose_to_pairs(xt, p0, 32)
        pltpu.sync_copy(p0.at[pl.ds(0, 32), :],
                        tabP.at[pl.ds(TAIL_C0 // 2, 32), :])


def _extract2(g_v, o_v):
    iot = lax.iota(jnp.int32, 16)
    rows = [lax.shift_right_logical(16 * q + iot, 3) for q in range(4)]
    cols = [lax.shift_left(lax.bitwise_and(16 * q + iot, 7), 7)
            for q in range(4)]
    for tt in range(2):
        ttv = jnp.full((16,), tt, jnp.int32)

        @plsc.parallel_loop(0, 128, unroll=8)
        def _cstep(c):
            cv = jnp.full((16,), 0, jnp.int32) + c
            for q in range(4):
                v = g_v[128 * tt + c, pl.ds(16 * q, 16)]
                plsc.store_scatter(o_v, [ttv, rows[q], cols[q] + cv], v)


def _k2_body(ids4, tabv, out5, ids_v, g0, g1, o0, o1,
             gs0, gs1, ws0, ws1):
    wid = _wid()
    pltpu.sync_copy(ids4.at[wid], ids_v)

    @plsc.parallel_loop(0, 25600, step=16, unroll=8)
    def _remap(i):
        v = ids_v[pl.ds(i, 16)]
        adj = jnp.where(v >= H1, 2 * H1 - 1, 0)
        ids_v[pl.ds(i, 16)] = lax.shift_left(v, 1) - adj

    gs = (g0, g1)
    os_ = (o0, o1)
    gss = (gs0, gs1)
    wss = (ws0, ws1)
    nk = T // 2

    def gather(k, b):
        kc = jnp.minimum(k, nk - 1)
        pltpu.async_copy(tabv.at[ids_v.at[pl.ds(256 * kc, 256)]],
                         gs[b], gss[b])

    def write(k, b):
        pltpu.async_copy(os_[b], out5.at[pl.ds(2 * k, 2), :, wid, :], wss[b])

    gather(0, 0)
    gather(1, 1)

    def step(k, b, wait_w):
        pltpu.make_async_copy(tabv.at[ids_v.at[pl.ds(0, 256)]], gs[b],
                              gss[b]).wait()
        if wait_w:
            pltpu.make_async_copy(os_[b], out5.at[pl.ds(0, 2), :, wid, :],
                                  wss[b]).wait()
        _extract2(gs[b], os_[b])
        gather(k + 2, b)
        write(k, b)

    step(0, 0, False)
    step(1, 1, False)

    def pairk(kk, carry):
        k0 = 2 * kk
        step(k0, 0, True)
        step(k0 + 1, 1, True)
        return carry

    lax.fori_loop(1, nk // 2, pairk, 0)

    for b in range(2):
        pltpu.make_async_copy(tabv.at[ids_v.at[pl.ds(0, 256)]], gs[b],
                              gss[b]).wait()
        pltpu.make_async_copy(os_[b], out5.at[pl.ds(0, 2), :, wid, :],
                              wss[b]).wait()


def _k1_tc_body(x1_ref, x2_ref, y_ref):
    y_ref[:, 0:64] = x1_ref[...].T
    y_ref[:, 64:128] = x2_ref[...].T


def _k1b_body(x1_ref, x2_ref, tin_ref, y_ref):
    del tin_ref
    y_ref[:, 0:64] = x1_ref[...].T
    y_ref[:, 64:128] = x2_ref[...].T


@jax.jit
def _embed(ids, table):
    tab_t = table.T

    k1 = pl.pallas_call(
        _k1_tc_body,
        grid=(H1 // 2048,),
        in_specs=[
            pl.BlockSpec((HIDDEN, 2048), lambda i: (0, i)),
            pl.BlockSpec((HIDDEN, 2048),
                         lambda i: (0, jnp.minimum(i + H1 // 2048,
                                                   VOCAB // 2048 - 1))),
        ],
        out_specs=pl.BlockSpec((2048, 128), lambda i: (i, 0)),
        out_shape=jax.ShapeDtypeStruct((H1, 128), jnp.float32),
    )
    tabP = k1(tab_t, tab_t)

    k1b = pl.pallas_call(
        _k1b_body,
        grid=(1,),
        in_specs=[
            pl.BlockSpec((HIDDEN, 1024), lambda i: (0, 497664 // 1024)),
            pl.BlockSpec((HIDDEN, 1024), lambda i: (0, 999424 // 1024)),
            pl.BlockSpec((8, 128), lambda i: (0, 0)),
        ],
        out_specs=pl.BlockSpec((1024, 128), lambda i: (497664 // 1024, 0)),
        out_shape=jax.ShapeDtypeStruct((H1, 128), jnp.float32),
        input_output_aliases={2: 0},
    )
    tabP = k1b(tab_t, tab_t, tabP)
    tabv = tabP.reshape(2 * H1, HIDDEN)

    ids4 = ids.T.reshape(T, NW, 128).transpose(1, 0, 2).reshape(NW, 25600)

    k2 = pl.kernel(
        _k2_body,
        mesh=_mesh(),
        out_type=jax.ShapeDtypeStruct((T, 8, 32, 1024), jnp.float32),
        scratch_types=[
            pltpu.VMEM((25600,), jnp.int32),
        ] + [pltpu.VMEM((256, HIDDEN), jnp.float32)] * 2
          + [pltpu.VMEM((2, 8, 1024), jnp.float32)] * 2
          + [pltpu.SemaphoreType.DMA] * 4,
        compiler_params=pltpu.CompilerParams(use_tc_tiling_on_sc=False,
                                             needs_layout_passes=False,
                                             disable_bounds_checks=True),
    )
    out5 = k2(ids4, tabv)
    out6 = out5.reshape(T, 8, 32, 8, 128)
    return jnp.transpose(out6, (2, 4, 0, 1, 3)).reshape(B, T, HIDDEN)


def kernel(input_ids, embed_tokens_weight):
    return _embed(input_ids.astype(jnp.int32), embed_tokens_weight)
